# baseline (speedup 1.0000x reference)
import functools

import jax
import jax.numpy as jnp
from jax import lax
from jax.experimental import pallas as pl
from jax.experimental.pallas import tpu as pltpu
from jax.experimental.pallas import tpu_sc as plsc

NC = 2
NS = 16
NW = NC * NS
CH = 128
NBUF = 4


def _ign(idx_ref):
    return plsc.Indices(idx_ref, ignored_value=-1)


@functools.lru_cache(maxsize=None)
def _build_sc_kernel(B, T, D):
    b_per_w = B // NW
    mrows = b_per_w // CH
    t_per_w = (T - B) // NW
    trows = t_per_w // CH
    ngroups = trows // NBUF

    mesh = plsc.VectorSubcoreMesh(core_axis_name="c", subcore_axis_name="s")

    idx_t = pltpu.VMEM((trows, CH), jnp.int32)
    idx_m = pltpu.VMEM((mrows, CH), jnp.int32)
    acc_t = pltpu.VMEM((CH, D), jnp.float32)

    @functools.partial(
        pl.kernel,
        out_type=(
            jax.ShapeDtypeStruct((B, D), jnp.float32),
            jax.ShapeDtypeStruct((NW, 1, D), jnp.float32),
        ),
        mesh=mesh,
        scratch_types=[
            idx_m, idx_m, idx_m,
            idx_t, idx_t, idx_t,
            pltpu.VMEM((b_per_w, D), jnp.float32),
            acc_t, acc_t, acc_t, acc_t,
            acc_t, acc_t, acc_t, acc_t,
            pltpu.VMEM((D,), jnp.float32),
            pltpu.VMEM((1, D), jnp.float32),
            pltpu.SemaphoreType.DMA, pltpu.SemaphoreType.DMA,
            pltpu.SemaphoreType.DMA, pltpu.SemaphoreType.DMA,
            pltpu.SemaphoreType.DMA, pltpu.SemaphoreType.DMA,
            pltpu.SemaphoreType.DMA, pltpu.SemaphoreType.DMA,
        ],
        compiler_params=pltpu.CompilerParams(use_tc_tiling_on_sc=False),
    )
    def sc_kernel(text_ma_hbm, text_mb_hbm, deps_m_hbm,
                  text_ta_hbm, text_tb_hbm, deps_t_hbm,
                  wta_hbm, wtb_hbm, wd_hbm, bias_hbm,
                  out_hbm, part_hbm,
                  idx_ma, idx_mb, idx_md, idx_ta, idx_tb, idx_td, outb,
                  a0, a1, a2, a3, b0, b1, b2, b3, bias_v, stage,
                  s0, s1, s2, s3, s4, s5, s6, s7):
        acc_a = (a0, a1, a2, a3)
        acc_b = (b0, b1, b2, b3)
        sem_a = (s0, s1, s2, s3)
        sem_b = (s4, s5, s6, s7)
        wid = lax.axis_index("s") * NC + lax.axis_index("c")

        pltpu.sync_copy(bias_hbm, bias_v)
        pltpu.sync_copy(text_ma_hbm.at[wid], idx_ma)
        pltpu.sync_copy(text_mb_hbm.at[wid], idx_mb)
        pltpu.sync_copy(deps_m_hbm.at[wid], idx_md)
        pltpu.sync_copy(text_ta_hbm.at[wid], idx_ta)
        pltpu.sync_copy(text_tb_hbm.at[wid], idx_tb)
        pltpu.sync_copy(deps_t_hbm.at[wid], idx_td)

        bv = bias_v[...]

        def init_main(i, carry):
            outb[i] = bv
            return carry

        lax.fori_loop(0, b_per_w, init_main, 0)

        zero = jnp.zeros((D,), jnp.float32)

        def init_acc(i, carry):
            for a in acc_a + acc_b:
                a[i] = zero
            return carry

        lax.fori_loop(0, CH, init_acc, 0)

        for src, idx in ((wta_hbm, idx_ma), (wtb_hbm, idx_mb),
                         (wd_hbm, idx_md)):
            for j in range(mrows):
                pltpu.async_copy(src.at[_ign(idx.at[j])],
                                 outb.at[pl.ds(j * CH, CH)], sem_a[j % NBUF],
                                 add=True)
            for j in range(mrows):
                pltpu.make_async_copy(src.at[_ign(idx.at[j])],
                                      outb.at[pl.ds(j * CH, CH)],
                                      sem_a[j % NBUF]).wait()
        pltpu.sync_copy(outb, out_hbm.at[pl.ds(wid * b_per_w, b_per_w)])

        def run_table(src_hbm, idx_ref, accs, sems):
            for b in range(NBUF):
                pltpu.async_copy(src_hbm.at[_ign(idx_ref.at[b])], accs[b],
                                 sems[b], add=True)

            def body(g, carry):
                for b in range(NBUF):
                    pltpu.make_async_copy(src_hbm.at[_ign(idx_ref.at[b])],
                                          accs[b], sems[b]).wait()
                    pltpu.async_copy(src_hbm.at[_ign(idx_ref.at[g * NBUF + b])],
                                     accs[b], sems[b], add=True)
                return carry

            lax.fori_loop(1, ngroups, body, 0)
            for b in range(NBUF):
                pltpu.make_async_copy(src_hbm.at[_ign(idx_ref.at[b])],
                                      accs[b], sems[b]).wait()

        run_table(wta_hbm, idx_ta, acc_a, sem_a)
        run_table(wtb_hbm, idx_tb, acc_b, sem_b)
        run_table(wd_hbm, idx_td, acc_a, sem_a)

        def red(i, carry):
            sa = (a0[i] + a1[i]) + (a2[i] + a3[i])
            sb = (b0[i] + b1[i]) + (b2[i] + b3[i])
            return carry + (sa + sb)

        total = lax.fori_loop(0, CH, red, jnp.zeros((D,), jnp.float32))
        stage[0] = total
        pltpu.sync_copy(stage, part_hbm.at[wid])

    return sc_kernel


def _fix_last_rows(partials_ref, last_ref, out_ref):
    s = jnp.sum(partials_ref[...], axis=0, keepdims=True)
    row = lax.broadcasted_iota(jnp.int32, (8, 1), 0)
    out_ref[...] = last_ref[...] + jnp.where(row == 7, s, 0.0)


def kernel(text, text_offsets, deps, deps_offsets, W_text, W_deps, bias):
    B = text_offsets.shape[0]
    T = text.shape[0]
    V = W_text.shape[0]
    D = W_text.shape[1]
    H = V // 2
    mrows = B // NW // CH
    trows = (T - B) // NW // CH

    text_i = text.astype(jnp.int32)
    deps_i = deps.astype(jnp.int32)
    text_a = jnp.where(text_i < H, text_i, -1)
    text_b = jnp.where(text_i >= H, text_i - H, -1)

    def pack_m(x):
        return x[:B].reshape(NW, mrows, CH)

    def pack_t(x):
        return x[B:].reshape(NW, trows, CH)

    sc_kernel = _build_sc_kernel(B, T, D)
    out_main, partials = sc_kernel(
        pack_m(text_a), pack_m(text_b), pack_m(deps_i),
        pack_t(text_a), pack_t(text_b), pack_t(deps_i),
        W_text[:H].astype(jnp.float32), W_text[H:].astype(jnp.float32),
        W_deps.astype(jnp.float32), bias.astype(jnp.float32))

    last_block = lax.slice(out_main, (B - 8, 0), (B, D))
    fixed = pl.pallas_call(
        _fix_last_rows,
        out_shape=jax.ShapeDtypeStruct((8, D), jnp.float32),
    )(partials.reshape(NW, D), last_block)
    return lax.dynamic_update_slice(out_main, fixed, (B - 8, 0))
